# baseline (speedup 1.0000x reference)
import functools

import jax
import jax.numpy as jnp
from jax import lax
from jax.experimental import pallas as pl
from jax.experimental.pallas import tpu as pltpu
from jax.experimental.pallas import tpu_sc as plsc

B, N, H = 512, 32, 100
HP = 128
G = 64
Q = 448
BT = B - Q
BPW = Q // 32

_LOG1P = (9.083788899692856e-08, 0.9999914545709757, -0.4998011631945514,
          0.3313340056740668, -0.2391907171688213, 0.16478349703876802,
          -0.0923137684185457, 0.03441859339279139, -0.006074877617407555)


def _tc_body(x_ref, mf_ref, me_ref, b1_ref, w2_ref, cst_ref, out_ref):
    x = x_ref[...]
    xf = x.reshape(G * N, 6)
    f = jnp.dot(xf, mf_ref[...], preferred_element_type=jnp.float32) + b1_ref[...]
    e = jnp.dot(xf, me_ref[...], preferred_element_type=jnp.float32)
    s_rows = []
    for g in range(G):
        fg = f[g * N:(g + 1) * N, :]
        eg = e[g * N:(g + 1) * N, :]
        sum_eg = jnp.sum(eg, axis=0, keepdims=True)
        t0 = jnp.abs(fg + eg[0:1, :])
        t1 = jnp.abs(fg + eg[1:2, :])
        for j in range(2, N, 2):
            t0 = t0 + jnp.abs(fg + eg[j:j + 1, :])
            t1 = t1 + jnp.abs(fg + eg[j + 1:j + 2, :])
        diag = fg + eg
        sg = (0.55 * (N * fg + sum_eg) + 0.45 * (t0 + t1)
              - (0.55 * diag + 0.45 * jnp.abs(diag)))
        s_rows.append(sg)
    s = jnp.concatenate(s_rows, axis=0)
    p = jnp.dot(s, w2_ref[...],
                preferred_element_type=jnp.float32) + cst_ref[...]
    sp = 0.1 * (jnp.maximum(p, 0.0) + jnp.log1p(jnp.exp(-jnp.abs(p))))
    upd = xf + 0.1 * p
    chan = jax.lax.broadcasted_iota(jnp.int32, (G * N, 6), 1)
    out_ref[...] = jnp.where(chan < 4, upd, sp).reshape(G, N, 6)


def _full16(v):
    return jnp.full((16,), v, jnp.int32)


def _sc_body(xt_ref, wt_ref, out_ref, xb, wtv, ft, et, ob, sem):
    wid = lax.axis_index("s") * 2 + lax.axis_index("c")
    base = wid * BPW
    pltpu.sync_copy(wt_ref, wtv)

    def batch_body(b, carry):
        pltpu.sync_copy(xt_ref.at[base + b], xb)
        xc = tuple(xb[c, pl.ds(o, 16)] for c in range(6) for o in (0, 16))

        def build_body(k, c0):
            kv = _full16(k)
            b1v = plsc.load_gather(wtv, [_full16(12), kv])
            mfc = [plsc.load_gather(wtv, [_full16(c), kv]) for c in range(6)]
            mec = [plsc.load_gather(wtv, [_full16(6 + c), kv]) for c in range(6)]
            f0 = ((xc[0] * mfc[0] + xc[2] * mfc[1]) + (xc[4] * mfc[2] + xc[6] * mfc[3])
                  + (xc[8] * mfc[4] + xc[10] * mfc[5] + b1v))
            f1 = ((xc[1] * mfc[0] + xc[3] * mfc[1]) + (xc[5] * mfc[2] + xc[7] * mfc[3])
                  + (xc[9] * mfc[4] + xc[11] * mfc[5] + b1v))
            e0 = ((xc[0] * mec[0] + xc[2] * mec[1]) + (xc[4] * mec[2] + xc[6] * mec[3])
                  + (xc[8] * mec[4] + xc[10] * mec[5]))
            e1 = ((xc[1] * mec[0] + xc[3] * mec[1]) + (xc[5] * mec[2] + xc[7] * mec[3])
                  + (xc[9] * mec[4] + xc[11] * mec[5]))
            ft[k, pl.ds(0, 16)] = f0
            ft[k, pl.ds(16, 16)] = f1
            et[k, pl.ds(0, 16)] = e0
            et[k, pl.ds(16, 16)] = e1
            return c0

        lax.fori_loop(0, H, build_body, 0)

        def k_body(k, pacc):
            kv = _full16(k)
            f0 = ft[k, pl.ds(0, 16)]
            f1 = ft[k, pl.ds(16, 16)]
            e0 = et[k, pl.ds(0, 16)]
            e1 = et[k, pl.ds(16, 16)]
            se = jnp.sum(e0) + jnp.sum(e1)
            acc = [None] * 8
            for j in range(N):
                ev = plsc.load_gather(et, [kv, _full16(j)])
                u0 = jnp.abs(f0 + ev)
                u1 = jnp.abs(f1 + ev)
                lane = j % 4
                if acc[lane] is None:
                    acc[lane] = u0
                    acc[4 + lane] = u1
                else:
                    acc[lane] = acc[lane] + u0
                    acc[4 + lane] = acc[4 + lane] + u1
            a0 = (acc[0] + acc[1]) + (acc[2] + acc[3])
            a1 = (acc[4] + acc[5]) + (acc[6] + acc[7])
            d0 = f0 + e0
            d1 = f1 + e1
            s0 = (0.55 * (N * f0 + se) + 0.45 * a0
                  - (0.55 * d0 + 0.45 * jnp.abs(d0)))
            s1 = (0.55 * (N * f1 + se) + 0.45 * a1
                  - (0.55 * d1 + 0.45 * jnp.abs(d1)))
            out = list(pacc)
            for c in range(6):
                wv = plsc.load_gather(wtv, [_full16(13 + c), kv])
                out[c] = pacc[c] + s0 * wv
                out[6 + c] = pacc[6 + c] + s1 * wv
            return tuple(out)

        zero = jnp.zeros((16,), jnp.float32)
        pacc = lax.fori_loop(0, H, k_body, (zero,) * 12)

        for c in range(6):
            cstv = plsc.load_gather(wtv, [_full16(19), _full16(c)])
            p0 = pacc[c] + cstv
            p1 = pacc[6 + c] + cstv
            if c < 4:
                o0 = xb[c, pl.ds(0, 16)] + 0.1 * p0
                o1 = xb[c, pl.ds(16, 16)] + 0.1 * p1
            else:
                def softplus(p):
                    z = jnp.exp(-jnp.abs(p))
                    l = jnp.full((16,), _LOG1P[8], jnp.float32)
                    for coef in reversed(_LOG1P[:8]):
                        l = l * z + coef
                    return jnp.maximum(p, 0.0) + l
                o0 = 0.1 * softplus(p0)
                o1 = 0.1 * softplus(p1)
            ob[c, pl.ds(0, 16)] = o0
            ob[c, pl.ds(16, 16)] = o1
        pltpu.sync_copy(ob, out_ref.at[base + b])
        return carry

    lax.fori_loop(0, BPW, batch_body, 0)


_sc_kernel = functools.partial(
    pl.kernel,
    mesh=plsc.VectorSubcoreMesh(core_axis_name="c", subcore_axis_name="s"),
    out_type=jax.ShapeDtypeStruct((Q, 6, 32), jnp.float32),
    scratch_types=[
        pltpu.VMEM((6, 32), jnp.float32),
        pltpu.VMEM((24, 104), jnp.float32),
        pltpu.VMEM((104, 32), jnp.float32),
        pltpu.VMEM((104, 32), jnp.float32),
        pltpu.VMEM((6, 32), jnp.float32),
        pltpu.SemaphoreType.DMA,
    ],
    compiler_params=pltpu.CompilerParams(needs_layout_passes=False),
)(_sc_body)


@jax.jit
def kernel(inp, W1, b1, gamma, beta, running_mean, running_var, W2, b2):
    f32 = jnp.float32
    inp = inp.astype(f32)
    s = gamma * jax.lax.rsqrt(running_var + 1e-5)
    t = beta - s * running_mean
    w2p = (W2 * s[None, :]).astype(f32)
    cst = (N - 1.0) * (W2 @ t + b2)
    wdy = W1[:, 8]
    wdx = W1[:, 9]
    mf = jnp.concatenate([wdy[:, None], wdx[:, None], W1[:, 0:4]], axis=1)
    me = jnp.concatenate([-wdy[:, None], -wdx[:, None], W1[:, 4:8]], axis=1)
    mf_p = jnp.zeros((6, HP), f32).at[:, :H].set(mf.T)
    me_p = jnp.zeros((6, HP), f32).at[:, :H].set(me.T)
    b1_p = jnp.zeros((1, HP), f32).at[:, :H].set(b1)
    w2_p = jnp.zeros((HP, 6), f32).at[:H, :].set(w2p.T)
    cst_p = cst.reshape(1, 6).astype(f32)

    out_tc = pl.pallas_call(
        _tc_body,
        grid=(BT // G,),
        in_specs=[
            pl.BlockSpec((G, N, 6), lambda g: (g, 0, 0)),
            pl.BlockSpec((6, HP), lambda g: (0, 0)),
            pl.BlockSpec((6, HP), lambda g: (0, 0)),
            pl.BlockSpec((1, HP), lambda g: (0, 0)),
            pl.BlockSpec((HP, 6), lambda g: (0, 0)),
            pl.BlockSpec((1, 6), lambda g: (0, 0)),
        ],
        out_specs=pl.BlockSpec((G, N, 6), lambda g: (g, 0, 0)),
        out_shape=jax.ShapeDtypeStruct((BT, N, 6), f32),
        compiler_params=pltpu.CompilerParams(
            dimension_semantics=("parallel",)),
    )(inp[:BT], mf_p, me_p, b1_p, w2_p, cst_p)

    wt = jnp.zeros((24, 104), f32)
    wt = wt.at[0:6, :H].set(mf.T)
    wt = wt.at[6:12, :H].set(me.T)
    wt = wt.at[12, :H].set(b1)
    wt = wt.at[13:19, :H].set(w2p)
    wt = wt.at[19, :6].set(cst)

    xt = inp[BT:].transpose(0, 2, 1)
    out_sc = _sc_kernel(xt, wt)
    return jnp.concatenate([out_tc, out_sc.transpose(0, 2, 1)], axis=0)


# speedup vs baseline: 2.7025x; 2.7025x over previous
import functools

import jax
import jax.numpy as jnp
from jax.experimental import pallas as pl
from jax.experimental.pallas import tpu as pltpu

B, N, H = 512, 32, 100
HP = 128
G = 64


def _body(x_ref, mf_ref, me_ref, b1_ref, w2_ref, cst_ref, out_ref):
    x = x_ref[...]
    xf = x.reshape(G * N, 6)
    f = jnp.dot(xf, mf_ref[...], preferred_element_type=jnp.float32) + b1_ref[...]
    e = jnp.dot(xf, me_ref[...], preferred_element_type=jnp.float32)
    fb16 = f.astype(jnp.bfloat16)
    eb16 = e.astype(jnp.bfloat16)
    s_rows = []
    for g in range(G):
        fg = f[g * N:(g + 1) * N, :]
        eg = e[g * N:(g + 1) * N, :]
        fgb = fb16[g * N:(g + 1) * N, :]
        egb = eb16[g * N:(g + 1) * N, :]
        sum_eg = jnp.sum(eg, axis=0, keepdims=True)
        t0 = jnp.abs(fgb + egb[0:1, :])
        t1 = jnp.abs(fgb + egb[1:2, :])
        for j in range(2, N, 2):
            t0 = t0 + jnp.abs(fgb + egb[j:j + 1, :])
            t1 = t1 + jnp.abs(fgb + egb[j + 1:j + 2, :])
        diag = fg + eg
        tt = (t0 + t1).astype(jnp.float32)
        sg = (0.55 * (N * fg + sum_eg) + 0.45 * tt
              - (0.55 * diag + 0.45 * jnp.abs(diag)))
        s_rows.append(sg)
    s = jnp.concatenate(s_rows, axis=0)
    p = jnp.dot(s, w2_ref[...],
                preferred_element_type=jnp.float32) + cst_ref[...]
    sp = 0.1 * (jnp.maximum(p, 0.0) + jnp.log1p(jnp.exp(-jnp.abs(p))))
    upd = xf + 0.1 * p
    chan = jax.lax.broadcasted_iota(jnp.int32, (G * N, 6), 1)
    out_ref[...] = jnp.where(chan < 4, upd, sp).reshape(G, N, 6)


@jax.jit
def kernel(inp, W1, b1, gamma, beta, running_mean, running_var, W2, b2):
    f32 = jnp.float32
    inp = inp.astype(f32)
    s = gamma * jax.lax.rsqrt(running_var + 1e-5)
    t = beta - s * running_mean
    w2p = (W2 * s[None, :]).astype(f32)
    cst = (N - 1.0) * (W2 @ t + b2)
    wdy = W1[:, 8]
    wdx = W1[:, 9]
    mf = jnp.concatenate([wdy[:, None], wdx[:, None], W1[:, 0:4]], axis=1)
    me = jnp.concatenate([-wdy[:, None], -wdx[:, None], W1[:, 4:8]], axis=1)
    mf_p = jnp.zeros((6, HP), f32).at[:, :H].set(mf.T)
    me_p = jnp.zeros((6, HP), f32).at[:, :H].set(me.T)
    b1_p = jnp.zeros((1, HP), f32).at[:, :H].set(b1)
    w2_p = jnp.zeros((HP, 6), f32).at[:H, :].set(w2p.T)
    cst_p = cst.reshape(1, 6).astype(f32)

    out = pl.pallas_call(
        _body,
        grid=(B // G,),
        in_specs=[
            pl.BlockSpec((G, N, 6), lambda g: (g, 0, 0)),
            pl.BlockSpec((6, HP), lambda g: (0, 0)),
            pl.BlockSpec((6, HP), lambda g: (0, 0)),
            pl.BlockSpec((1, HP), lambda g: (0, 0)),
            pl.BlockSpec((HP, 6), lambda g: (0, 0)),
            pl.BlockSpec((1, 6), lambda g: (0, 0)),
        ],
        out_specs=pl.BlockSpec((G, N, 6), lambda g: (g, 0, 0)),
        out_shape=jax.ShapeDtypeStruct((B, N, 6), f32),
        compiler_params=pltpu.CompilerParams(
            dimension_semantics=("parallel",)),
    )(inp, mf_p, me_p, b1_p, w2_p, cst_p)
    return out


# speedup vs baseline: 2.8032x; 1.0373x over previous
import functools

import jax
import jax.numpy as jnp
from jax.experimental import pallas as pl
from jax.experimental.pallas import tpu as pltpu

B, N, H = 512, 32, 100
HP = 128
G = 64


def _body(x_ref, mf_ref, me_ref, b1_ref, w2_ref, w2b_ref, cst_ref, out_ref):
    x = x_ref[...]
    xf = x.reshape(G * N, 6)
    f = jnp.dot(xf, mf_ref[...], preferred_element_type=jnp.float32) + b1_ref[...]
    e = jnp.dot(xf, me_ref[...], preferred_element_type=jnp.float32)
    bf16 = jnp.bfloat16
    fb16 = f.astype(bf16)
    eb16 = e.astype(bf16)
    s_rows = []
    for g in range(G):
        fgb = fb16[g * N:(g + 1) * N, :]
        egb = eb16[g * N:(g + 1) * N, :]
        t0 = jnp.abs(fgb + egb[0:1, :])
        t1 = jnp.abs(fgb + egb[1:2, :])
        for j in range(2, N, 2):
            t0 = t0 + jnp.abs(fgb + egb[j:j + 1, :])
            t1 = t1 + jnp.abs(fgb + egb[j + 1:j + 2, :])
        diag = fgb + egb
        sg = (bf16(0.45) * (t0 + t1)
              - (bf16(0.55) * diag + bf16(0.45) * jnp.abs(diag)))
        s_rows.append(sg)
    s = jnp.concatenate(s_rows, axis=0)
    pf = jnp.dot(f, w2_ref[...], preferred_element_type=jnp.float32)
    pe = jnp.dot(e, w2_ref[...], preferred_element_type=jnp.float32)
    se = jnp.sum(pe.reshape(G, N, 6), axis=1, keepdims=True)
    pt = jnp.dot(s, w2b_ref[...], preferred_element_type=jnp.float32)
    p = (pt + 0.55 * (N * pf + jnp.broadcast_to(se, (G, N, 6)).reshape(G * N, 6))
         + cst_ref[...])
    sp = 0.1 * (jnp.maximum(p, 0.0) + jnp.log1p(jnp.exp(-jnp.abs(p))))
    upd = xf + 0.1 * p
    chan = jax.lax.broadcasted_iota(jnp.int32, (G * N, 6), 1)
    out_ref[...] = jnp.where(chan < 4, upd, sp).reshape(G, N, 6)


@jax.jit
def kernel(inp, W1, b1, gamma, beta, running_mean, running_var, W2, b2):
    f32 = jnp.float32
    inp = inp.astype(f32)
    s = gamma * jax.lax.rsqrt(running_var + 1e-5)
    t = beta - s * running_mean
    w2p = (W2 * s[None, :]).astype(f32)
    cst = (N - 1.0) * (W2 @ t + b2)
    wdy = W1[:, 8]
    wdx = W1[:, 9]
    mf = jnp.concatenate([wdy[:, None], wdx[:, None], W1[:, 0:4]], axis=1)
    me = jnp.concatenate([-wdy[:, None], -wdx[:, None], W1[:, 4:8]], axis=1)
    mf_p = jnp.zeros((6, HP), f32).at[:, :H].set(mf.T)
    me_p = jnp.zeros((6, HP), f32).at[:, :H].set(me.T)
    b1_p = jnp.zeros((1, HP), f32).at[:, :H].set(b1)
    w2_p = jnp.zeros((HP, 6), f32).at[:H, :].set(w2p.T)
    w2b_p = w2_p.astype(jnp.bfloat16)
    cst_p = cst.reshape(1, 6).astype(f32)

    out = pl.pallas_call(
        _body,
        grid=(B // G,),
        in_specs=[
            pl.BlockSpec((G, N, 6), lambda g: (g, 0, 0)),
            pl.BlockSpec((6, HP), lambda g: (0, 0)),
            pl.BlockSpec((6, HP), lambda g: (0, 0)),
            pl.BlockSpec((1, HP), lambda g: (0, 0)),
            pl.BlockSpec((HP, 6), lambda g: (0, 0)),
            pl.BlockSpec((HP, 6), lambda g: (0, 0)),
            pl.BlockSpec((1, 6), lambda g: (0, 0)),
        ],
        out_specs=pl.BlockSpec((G, N, 6), lambda g: (g, 0, 0)),
        out_shape=jax.ShapeDtypeStruct((B, N, 6), f32),
        compiler_params=pltpu.CompilerParams(
            dimension_semantics=("parallel",)),
    )(inp, mf_p, me_p, b1_p, w2_p, w2b_p, cst_p)
    return out


# speedup vs baseline: 2.8499x; 1.0167x over previous
import functools

import jax
import jax.numpy as jnp
from jax.experimental import pallas as pl
from jax.experimental.pallas import tpu as pltpu

B, N, H = 512, 32, 100
HP = 128
G = 64


def _body(x_ref, mf_ref, me_ref, b1_ref, w2f_ref, w2e_ref, w2b_ref, cst_ref, out_ref):
    x = x_ref[...]
    xf = x.reshape(G * N, 6)
    f = jnp.dot(xf, mf_ref[...], preferred_element_type=jnp.float32) + b1_ref[...]
    e = jnp.dot(xf, me_ref[...], preferred_element_type=jnp.float32)
    bf16 = jnp.bfloat16
    fb16 = f.astype(bf16)
    eb16 = e.astype(bf16)
    s_rows = []
    for g in range(G):
        fgb = fb16[g * N:(g + 1) * N, :]
        egb = eb16[g * N:(g + 1) * N, :]
        t = [jnp.abs(fgb + egb[j:j + 1, :]) for j in range(4)]
        for j in range(4, N, 4):
            for r in range(4):
                t[r] = t[r] + jnp.abs(fgb + egb[j + r:j + r + 1, :])
        diag = fgb + egb
        sg = (bf16(0.45) * ((t[0] + t[1]) + (t[2] + t[3]))
              - (bf16(0.55) * diag + bf16(0.45) * jnp.abs(diag)))
        s_rows.append(sg)
    s = jnp.concatenate(s_rows, axis=0)
    pf = jnp.dot(f, w2f_ref[...], preferred_element_type=jnp.float32)
    pe = jnp.dot(e, w2e_ref[...], preferred_element_type=jnp.float32)
    se = (jnp.sum(pe.reshape(G, N, 6), axis=1, keepdims=True)
          + cst_ref[...])
    pt = jnp.dot(s, w2b_ref[...], preferred_element_type=jnp.float32)
    p = ((pt + pf)
         + jnp.broadcast_to(se, (G, N, 6)).reshape(G * N, 6))
    sp = 0.1 * (jnp.maximum(p, 0.0) + jnp.log1p(jnp.exp(-jnp.abs(p))))
    upd = xf + 0.1 * p
    chan = jax.lax.broadcasted_iota(jnp.int32, (G * N, 6), 1)
    out_ref[...] = jnp.where(chan < 4, upd, sp).reshape(G, N, 6)


@jax.jit
def kernel(inp, W1, b1, gamma, beta, running_mean, running_var, W2, b2):
    f32 = jnp.float32
    inp = inp.astype(f32)
    s = gamma * jax.lax.rsqrt(running_var + 1e-5)
    t = beta - s * running_mean
    w2p = (W2 * s[None, :]).astype(f32)
    cst = (N - 1.0) * (W2 @ t + b2)
    wdy = W1[:, 8]
    wdx = W1[:, 9]
    mf = jnp.concatenate([wdy[:, None], wdx[:, None], W1[:, 0:4]], axis=1)
    me = jnp.concatenate([-wdy[:, None], -wdx[:, None], W1[:, 4:8]], axis=1)
    mf_p = jnp.zeros((6, HP), f32).at[:, :H].set(mf.T)
    me_p = jnp.zeros((6, HP), f32).at[:, :H].set(me.T)
    b1_p = jnp.zeros((1, HP), f32).at[:, :H].set(b1)
    w2_p = jnp.zeros((HP, 6), f32).at[:H, :].set(w2p.T)
    w2f_p = 0.55 * N * w2_p
    w2e_p = 0.55 * w2_p
    w2b_p = w2_p.astype(jnp.bfloat16)
    cst_p = cst.reshape(1, 1, 6).astype(f32)

    out = pl.pallas_call(
        _body,
        grid=(B // G,),
        in_specs=[
            pl.BlockSpec((G, N, 6), lambda g: (g, 0, 0)),
            pl.BlockSpec((6, HP), lambda g: (0, 0)),
            pl.BlockSpec((6, HP), lambda g: (0, 0)),
            pl.BlockSpec((1, HP), lambda g: (0, 0)),
            pl.BlockSpec((HP, 6), lambda g: (0, 0)),
            pl.BlockSpec((HP, 6), lambda g: (0, 0)),
            pl.BlockSpec((HP, 6), lambda g: (0, 0)),
            pl.BlockSpec((1, 1, 6), lambda g: (0, 0, 0)),
        ],
        out_specs=pl.BlockSpec((G, N, 6), lambda g: (g, 0, 0)),
        out_shape=jax.ShapeDtypeStruct((B, N, 6), f32),
        compiler_params=pltpu.CompilerParams(
            dimension_semantics=("parallel",)),
    )(inp, mf_p, me_p, b1_p, w2f_p, w2e_p, w2b_p, cst_p)
    return out


# speedup vs baseline: 2.9415x; 1.0322x over previous
import functools

import jax
import jax.numpy as jnp
from jax.experimental import pallas as pl
from jax.experimental.pallas import tpu as pltpu

B, N, H = 512, 32, 100
HP = 128
G = 64


def _body(x_ref, mf_ref, me_ref, b1_ref, w2f_ref, w2e_ref, w2b_ref, cst_ref, out_ref):
    x = x_ref[...]
    xf = x.reshape(G * N, 6)
    f = jnp.dot(xf, mf_ref[...], preferred_element_type=jnp.float32) + b1_ref[...]
    e = jnp.dot(xf, me_ref[...], preferred_element_type=jnp.float32)
    bf16 = jnp.bfloat16
    fb16 = f.astype(bf16)
    eb16 = e.astype(bf16)
    s_rows = []
    for g in range(G):
        fgb = fb16[g * N:(g + 1) * N, :]
        egb = eb16[g * N:(g + 1) * N, :]
        t = [jnp.abs(fgb + egb[j:j + 1, :]) for j in range(4)]
        for j in range(4, N, 4):
            for r in range(4):
                t[r] = t[r] + jnp.abs(fgb + egb[j + r:j + r + 1, :])
        diag = fgb + egb
        sg = (bf16(0.45) * ((t[0] + t[1]) + (t[2] + t[3]))
              - (bf16(0.55) * diag + bf16(0.45) * jnp.abs(diag)))
        s_rows.append(sg)
    s = jnp.concatenate(s_rows, axis=0)
    pf = jnp.dot(f, w2f_ref[...], preferred_element_type=jnp.float32)
    pe = jnp.dot(e, w2e_ref[...], preferred_element_type=jnp.float32)
    se = (jnp.sum(pe.reshape(G, N, 6), axis=1, keepdims=True)
          + cst_ref[...])
    pt = jnp.dot(s, w2b_ref[...], preferred_element_type=jnp.float32)
    p = ((pt + pf)
         + jnp.broadcast_to(se, (G, N, 6)).reshape(G * N, 6))
    pt_ = jnp.transpose(p)
    xft = jnp.transpose(xf)
    sp = 0.1 * (jnp.maximum(pt_, 0.0) + jnp.log1p(jnp.exp(-jnp.abs(pt_))))
    upd = xft + 0.1 * pt_
    chan = jax.lax.broadcasted_iota(jnp.int32, (6, G * N), 0)
    out_t = jnp.where(chan < 4, upd, sp)
    out_ref[...] = jnp.transpose(out_t).reshape(G, N, 6)


@jax.jit
def kernel(inp, W1, b1, gamma, beta, running_mean, running_var, W2, b2):
    f32 = jnp.float32
    inp = inp.astype(f32)
    s = gamma * jax.lax.rsqrt(running_var + 1e-5)
    t = beta - s * running_mean
    w2p = (W2 * s[None, :]).astype(f32)
    cst = (N - 1.0) * (W2 @ t + b2)
    wdy = W1[:, 8]
    wdx = W1[:, 9]
    mf = jnp.concatenate([wdy[:, None], wdx[:, None], W1[:, 0:4]], axis=1)
    me = jnp.concatenate([-wdy[:, None], -wdx[:, None], W1[:, 4:8]], axis=1)
    mf_p = jnp.zeros((6, HP), f32).at[:, :H].set(mf.T)
    me_p = jnp.zeros((6, HP), f32).at[:, :H].set(me.T)
    b1_p = jnp.zeros((1, HP), f32).at[:, :H].set(b1)
    w2_p = jnp.zeros((HP, 6), f32).at[:H, :].set(w2p.T)
    w2f_p = 0.55 * N * w2_p
    w2e_p = 0.55 * w2_p
    w2b_p = w2_p.astype(jnp.bfloat16)
    cst_p = cst.reshape(1, 1, 6).astype(f32)

    out = pl.pallas_call(
        _body,
        grid=(B // G,),
        in_specs=[
            pl.BlockSpec((G, N, 6), lambda g: (g, 0, 0)),
            pl.BlockSpec((6, HP), lambda g: (0, 0)),
            pl.BlockSpec((6, HP), lambda g: (0, 0)),
            pl.BlockSpec((1, HP), lambda g: (0, 0)),
            pl.BlockSpec((HP, 6), lambda g: (0, 0)),
            pl.BlockSpec((HP, 6), lambda g: (0, 0)),
            pl.BlockSpec((HP, 6), lambda g: (0, 0)),
            pl.BlockSpec((1, 1, 6), lambda g: (0, 0, 0)),
        ],
        out_specs=pl.BlockSpec((G, N, 6), lambda g: (g, 0, 0)),
        out_shape=jax.ShapeDtypeStruct((B, N, 6), f32),
        compiler_params=pltpu.CompilerParams(
            dimension_semantics=("parallel",)),
    )(inp, mf_p, me_p, b1_p, w2f_p, w2e_p, w2b_p, cst_p)
    return out


# speedup vs baseline: 2.9713x; 1.0101x over previous
import functools

import jax
import jax.numpy as jnp
from jax.experimental import pallas as pl
from jax.experimental.pallas import tpu as pltpu

B, N, H = 512, 32, 100
HP = 128
G = 128


def _body(x_ref, mf_ref, me_ref, b1_ref, w2f_ref, w2e_ref, w2b_ref, cst_ref, out_ref):
    x = x_ref[...]
    xf = x.reshape(G * N, 6)
    f = jnp.dot(xf, mf_ref[...], preferred_element_type=jnp.float32) + b1_ref[...]
    e = jnp.dot(xf, me_ref[...], preferred_element_type=jnp.float32)
    bf16 = jnp.bfloat16
    fb16 = f.astype(bf16)
    eb16 = e.astype(bf16)
    s_rows = []
    for g in range(G):
        fgb = fb16[g * N:(g + 1) * N, :]
        egb = eb16[g * N:(g + 1) * N, :]
        t = [jnp.abs(fgb + egb[j:j + 1, :]) for j in range(4)]
        for j in range(4, N, 4):
            for r in range(4):
                t[r] = t[r] + jnp.abs(fgb + egb[j + r:j + r + 1, :])
        diag = fgb + egb
        sg = (bf16(0.45) * ((t[0] + t[1]) + (t[2] + t[3]))
              - (bf16(0.55) * diag + bf16(0.45) * jnp.abs(diag)))
        s_rows.append(sg)
    s = jnp.concatenate(s_rows, axis=0)
    pf = jnp.dot(f, w2f_ref[...], preferred_element_type=jnp.float32)
    pe = jnp.dot(e, w2e_ref[...], preferred_element_type=jnp.float32)
    se = (jnp.sum(pe.reshape(G, N, 6), axis=1, keepdims=True)
          + cst_ref[...])
    pt = jnp.dot(s, w2b_ref[...], preferred_element_type=jnp.float32)
    p = ((pt + pf)
         + jnp.broadcast_to(se, (G, N, 6)).reshape(G * N, 6))
    pt_ = jnp.transpose(p)
    xft = jnp.transpose(xf)
    sp = 0.1 * (jnp.maximum(pt_, 0.0) + jnp.log1p(jnp.exp(-jnp.abs(pt_))))
    upd = xft + 0.1 * pt_
    chan = jax.lax.broadcasted_iota(jnp.int32, (6, G * N), 0)
    out_t = jnp.where(chan < 4, upd, sp)
    out_ref[...] = jnp.transpose(out_t).reshape(G, N, 6)


@jax.jit
def kernel(inp, W1, b1, gamma, beta, running_mean, running_var, W2, b2):
    f32 = jnp.float32
    inp = inp.astype(f32)
    s = gamma * jax.lax.rsqrt(running_var + 1e-5)
    t = beta - s * running_mean
    w2p = (W2 * s[None, :]).astype(f32)
    cst = (N - 1.0) * (W2 @ t + b2)
    wdy = W1[:, 8]
    wdx = W1[:, 9]
    mf = jnp.concatenate([wdy[:, None], wdx[:, None], W1[:, 0:4]], axis=1)
    me = jnp.concatenate([-wdy[:, None], -wdx[:, None], W1[:, 4:8]], axis=1)
    mf_p = jnp.zeros((6, HP), f32).at[:, :H].set(mf.T)
    me_p = jnp.zeros((6, HP), f32).at[:, :H].set(me.T)
    b1_p = jnp.zeros((1, HP), f32).at[:, :H].set(b1)
    w2_p = jnp.zeros((HP, 6), f32).at[:H, :].set(w2p.T)
    w2f_p = 0.55 * N * w2_p
    w2e_p = 0.55 * w2_p
    w2b_p = w2_p.astype(jnp.bfloat16)
    cst_p = cst.reshape(1, 1, 6).astype(f32)

    out = pl.pallas_call(
        _body,
        grid=(B // G,),
        in_specs=[
            pl.BlockSpec((G, N, 6), lambda g: (g, 0, 0)),
            pl.BlockSpec((6, HP), lambda g: (0, 0)),
            pl.BlockSpec((6, HP), lambda g: (0, 0)),
            pl.BlockSpec((1, HP), lambda g: (0, 0)),
            pl.BlockSpec((HP, 6), lambda g: (0, 0)),
            pl.BlockSpec((HP, 6), lambda g: (0, 0)),
            pl.BlockSpec((HP, 6), lambda g: (0, 0)),
            pl.BlockSpec((1, 1, 6), lambda g: (0, 0, 0)),
        ],
        out_specs=pl.BlockSpec((G, N, 6), lambda g: (g, 0, 0)),
        out_shape=jax.ShapeDtypeStruct((B, N, 6), f32),
        compiler_params=pltpu.CompilerParams(
            dimension_semantics=("parallel",)),
    )(inp, mf_p, me_p, b1_p, w2f_p, w2e_p, w2b_p, cst_p)
    return out
